# baseline (speedup 1.0000x reference)
import functools

import numpy as np

import jax
import jax.numpy as jnp
from jax import lax
from jax.experimental import pallas as pl
from jax.experimental.pallas import tpu as pltpu
from jax.experimental.pallas import tpu_sc as plsc

N_NODES = 10000
N_EDGES = 320000
IN_DIM = 128
EMB = 128
EDGE_DIM = 16
N_LAYERS = 3
N_GRAPHS = 64
N_CLASS = 10

def _make_perm():
    p = np.zeros(EMB, np.int32)
    for k in range(EMB // 32):
        for t in range(16):
            p[32 * k + 2 * t] = 32 * k + t
            p[32 * k + 2 * t + 1] = 32 * k + 16 + t
    return p


_PERM = _make_perm()

SC_CORES = 2
SC_SUBCORES = 16
HALF = EMB // SC_CORES
ROWS_PER_TILE = N_NODES // SC_SUBCORES
EDGES_PER_TILE = N_EDGES // SC_SUBCORES
SUB = 100
KSUB = 2
SUPER = SUB * KSUB
N_SUPER = EDGES_PER_TILE // SUPER
IDX_ROWS = N_EDGES // SUB


def _encode_body(x_ref, w_ref, wp_ref, o_ref, os_ref):
    x = x_ref[...]
    o_ref[...] = jnp.dot(x, w_ref[...], preferred_element_type=jnp.float32)
    os_ref[...] = jnp.dot(x, wp_ref[...],
                          preferred_element_type=jnp.float32
                          ).astype(jnp.bfloat16)


def _encode(x, w_in, w_in_p, interpret=False):
    bm = 2000
    return pl.pallas_call(
        _encode_body,
        grid=(N_NODES // bm,),
        in_specs=[
            pl.BlockSpec((bm, IN_DIM), lambda i: (i, 0)),
            pl.BlockSpec((IN_DIM, EMB), lambda i: (0, 0)),
            pl.BlockSpec((IN_DIM, EMB), lambda i: (0, 0)),
        ],
        out_specs=[
            pl.BlockSpec((bm, EMB), lambda i: (i, 0)),
            pl.BlockSpec((bm, EMB), lambda i: (i, 0)),
        ],
        out_shape=[
            jax.ShapeDtypeStruct((N_NODES, EMB), jnp.float32),
            jax.ShapeDtypeStruct((N_NODES, EMB), jnp.bfloat16),
        ],
        interpret=interpret,
    )(x, w_in, w_in_p)


def _edge_emb_body(ea_ref, w_ref, o0_ref, o1_ref, o2_ref):
    ea = ea_ref[...]
    for l, o in enumerate((o0_ref, o1_ref, o2_ref)):
        o[...] = jnp.dot(ea, w_ref[l], preferred_element_type=jnp.float32
                         ).astype(jnp.bfloat16)


def _edge_emb_all(edge_attr, w_edge, interpret=False):
    bm = 4000
    out = jax.ShapeDtypeStruct((N_EDGES, EMB), jnp.bfloat16)
    return pl.pallas_call(
        _edge_emb_body,
        grid=(N_EDGES // bm,),
        in_specs=[
            pl.BlockSpec((bm, EDGE_DIM), lambda i: (i, 0)),
            pl.BlockSpec((N_LAYERS, EDGE_DIM, EMB), lambda i: (0, 0, 0)),
        ],
        out_specs=[pl.BlockSpec((bm, EMB), lambda i: (i, 0))] * 3,
        out_shape=[out, out, out],
        interpret=interpret,
    )(edge_attr, w_edge)


def _mlp_body(h_ref, agg_ref, eps_ref, w1_ref, b1_ref, w2_ref, b2_ref,
              w2p_ref, b2p_ref, o_ref, os_ref, *, relu_out):
    z = (1.0 + eps_ref[0, 0]) * h_ref[...] + agg_ref[...]
    a = jnp.maximum(jnp.dot(z, w1_ref[...],
                            preferred_element_type=jnp.float32)
                    + b1_ref[...], 0.0)
    o = jnp.dot(a, w2_ref[...], preferred_element_type=jnp.float32) + b2_ref[...]
    op = (jnp.dot(a, w2p_ref[...], preferred_element_type=jnp.float32)
          + b2p_ref[...])
    if relu_out:
        o = jnp.maximum(o, 0.0)
        op = jnp.maximum(op, 0.0)
    o_ref[...] = o
    os_ref[...] = op.astype(jnp.bfloat16)


def _mlp(h, agg, eps_l, w1, b1, w2, b2, w2p, b2p, relu_out, interpret=False):
    bm = 2000
    return pl.pallas_call(
        functools.partial(_mlp_body, relu_out=relu_out),
        grid=(N_NODES // bm,),
        in_specs=[
            pl.BlockSpec((bm, EMB), lambda i: (i, 0)),
            pl.BlockSpec((bm, EMB), lambda i: (i, 0)),
            pl.BlockSpec((1, 1), lambda i: (0, 0)),
            pl.BlockSpec((EMB, 2 * EMB), lambda i: (0, 0)),
            pl.BlockSpec((1, 2 * EMB), lambda i: (0, 0)),
            pl.BlockSpec((2 * EMB, EMB), lambda i: (0, 0)),
            pl.BlockSpec((1, EMB), lambda i: (0, 0)),
            pl.BlockSpec((2 * EMB, EMB), lambda i: (0, 0)),
            pl.BlockSpec((1, EMB), lambda i: (0, 0)),
        ],
        out_specs=[
            pl.BlockSpec((bm, EMB), lambda i: (i, 0)),
            pl.BlockSpec((bm, EMB), lambda i: (i, 0)),
        ],
        out_shape=[
            jax.ShapeDtypeStruct((N_NODES, EMB), jnp.float32),
            jax.ShapeDtypeStruct((N_NODES, EMB), jnp.bfloat16),
        ],
        interpret=interpret,
    )(h, agg, eps_l.reshape(1, 1), w1, b1.reshape(1, -1), w2,
      b2.reshape(1, -1), w2p, b2p.reshape(1, -1))


def _pool_head_body(h_ref, bv_ref, wp_ref, bp_ref, pred_ref, hg_ref):
    bv = bv_ref[...]
    gids = lax.broadcasted_iota(jnp.int32, (N_GRAPHS, N_NODES), 0)
    oh = (gids == bv).astype(jnp.float32)
    sums = jnp.dot(oh, h_ref[...], preferred_element_type=jnp.float32)
    counts = jnp.sum(oh, axis=1, keepdims=True)
    hg = sums / jnp.maximum(counts, 1.0)
    hg_ref[...] = hg
    pred_ref[...] = jnp.dot(hg, wp_ref[...],
                            preferred_element_type=jnp.float32) + bp_ref[...]


def _pool_head(h, batch_vec, w_pred, b_pred, interpret=False):
    return pl.pallas_call(
        _pool_head_body,
        in_specs=[
            pl.BlockSpec((N_NODES, EMB), lambda: (0, 0)),
            pl.BlockSpec((1, N_NODES), lambda: (0, 0)),
            pl.BlockSpec((EMB, N_CLASS), lambda: (0, 0)),
            pl.BlockSpec((1, N_CLASS), lambda: (0, 0)),
        ],
        out_specs=[
            pl.BlockSpec((N_GRAPHS, N_CLASS), lambda: (0, 0)),
            pl.BlockSpec((N_GRAPHS, EMB), lambda: (0, 0)),
        ],
        out_shape=[
            jax.ShapeDtypeStruct((N_GRAPHS, N_CLASS), jnp.float32),
            jax.ShapeDtypeStruct((N_GRAPHS, EMB), jnp.float32),
        ],
        interpret=interpret,
    )(h, batch_vec.reshape(1, -1), w_pred, b_pred.reshape(1, -1))


def _gin_edge_body(h_hbm, e_hbm, src_hbm, dst_hbm, zeros_hbm, out_hbm,
                   h_sh, agg_sh,
                   src0, dst0, e0, src1, dst1, e1, ga, gb, ma, mb,
                   ldsem0, ldsem1, gsa, gsb, sca, scb):
    cid = lax.axis_index("c")
    sid = lax.axis_index("s")
    col0 = cid * HALF
    r0 = sid * ROWS_PER_TILE
    row_base = sid * (EDGES_PER_TILE // SUB)
    e_base = sid * EDGES_PER_TILE

    def loads(c, srcb, dstb, eb, sem):
        rb = row_base + c * KSUB
        eo = e_base + c * SUPER
        pltpu.async_copy(src_hbm.at[pl.ds(rb, KSUB), :], srcb, sem)
        pltpu.async_copy(dst_hbm.at[pl.ds(rb, KSUB), :], dstb, sem)
        pltpu.async_copy(e_hbm.at[pl.ds(eo, SUPER), pl.ds(col0, HALF)], eb,
                         sem)

    def wait_loads(srcb, dstb, eb, sem):
        pltpu.make_async_copy(src_hbm.at[pl.ds(row_base, KSUB), :], srcb,
                              sem).wait()
        pltpu.make_async_copy(dst_hbm.at[pl.ds(row_base, KSUB), :], dstb,
                              sem).wait()
        pltpu.make_async_copy(
            e_hbm.at[pl.ds(e_base, SUPER), pl.ds(col0, HALF)], eb, sem).wait()

    def compute(eb, k, g, m):
        @plsc.parallel_loop(0, SUB, step=1, unroll=4)
        def _(j):
            row = k * SUB + j
            for v in range(HALF // 32):
                sl = pl.ds(v * 32, 32)
                msg = jnp.maximum(g[j, sl] + eb[row, sl],
                                  jnp.bfloat16(0.0))
                lo, hi = plsc.unpack(msg, format=plsc.PackFormat.INTERLEAVED)
                m[j, pl.ds(v * 32, 16)] = lo
                m[j, pl.ds(v * 32 + 16, 16)] = hi

    def process(srcb, dstb, eb):
        pltpu.async_copy(h_sh.at[srcb.at[0]], ga, gsa)
        pltpu.async_copy(h_sh.at[srcb.at[1]], gb, gsb)
        pltpu.make_async_copy(h_sh.at[srcb.at[0]], ga, gsa).wait()
        compute(eb, 0, ga, ma)
        pltpu.async_copy(ma, agg_sh.at[dstb.at[0]], sca, add=True)
        pltpu.make_async_copy(h_sh.at[srcb.at[1]], gb, gsb).wait()
        compute(eb, 1, gb, mb)
        pltpu.async_copy(mb, agg_sh.at[dstb.at[1]], scb, add=True)
        pltpu.make_async_copy(ma, agg_sh.at[dstb.at[0]], sca).wait()
        pltpu.make_async_copy(mb, agg_sh.at[dstb.at[1]], scb).wait()

    pltpu.sync_copy(h_hbm.at[pl.ds(r0, ROWS_PER_TILE), pl.ds(col0, HALF)],
                    h_sh.at[pl.ds(r0, ROWS_PER_TILE), :])
    pltpu.sync_copy(zeros_hbm, agg_sh.at[pl.ds(r0, ROWS_PER_TILE), :])

    loads(0, src0, dst0, e0, ldsem0)
    loads(1, src1, dst1, e1, ldsem1)
    plsc.subcore_barrier()

    def super_body(i, carry):
        wait_loads(src0, dst0, e0, ldsem0)
        process(src0, dst0, e0)

        @pl.when(i < N_SUPER // 2 - 1)
        def _():
            loads(2 * i + 2, src0, dst0, e0, ldsem0)

        wait_loads(src1, dst1, e1, ldsem1)
        process(src1, dst1, e1)

        @pl.when(i < N_SUPER // 2 - 1)
        def _():
            loads(2 * i + 3, src1, dst1, e1, ldsem1)

        return carry

    lax.fori_loop(0, N_SUPER // 2, super_body, 0)
    plsc.subcore_barrier()

    pltpu.sync_copy(agg_sh.at[pl.ds(r0, ROWS_PER_TILE), :],
                    out_hbm.at[pl.ds(r0, ROWS_PER_TILE), pl.ds(col0, HALF)])


def _gin_edge(h, e_emb, src2d, dst2d, zeros, interpret=False):
    mesh = plsc.VectorSubcoreMesh(core_axis_name="c", subcore_axis_name="s",
                                  num_cores=SC_CORES,
                                  num_subcores=SC_SUBCORES)
    idx_t = pltpu.VMEM((KSUB, SUB), jnp.int32)
    e_t = pltpu.VMEM((SUPER, HALF), jnp.bfloat16)
    g_t = pltpu.VMEM((SUB, HALF), jnp.bfloat16)
    m_t = pltpu.VMEM((SUB, HALF), jnp.float32)
    sem = pltpu.SemaphoreType.DMA
    f = pl.kernel(
        _gin_edge_body,
        out_type=jax.ShapeDtypeStruct((N_NODES, EMB), jnp.float32),
        mesh=mesh,
        scratch_types=[
            pltpu.VMEM_SHARED((N_NODES, HALF), jnp.bfloat16),
            pltpu.VMEM_SHARED((N_NODES, HALF), jnp.float32),
            idx_t, idx_t, e_t,
            idx_t, idx_t, e_t,
            g_t, g_t, m_t, m_t,
            sem, sem, sem, sem, sem, sem,
        ],
        compiler_params=pltpu.CompilerParams(use_tc_tiling_on_sc=False,
                                             needs_layout_passes=False),
        interpret=interpret,
    )
    return f(h, e_emb, src2d, dst2d, zeros)


def kernel(x, edge_index, edge_attr, batch_vec, W_in, W_edge, W1, b1, W2, b2,
           eps, W_pred, b_pred):
    src = edge_index[0].astype(jnp.int32).reshape(IDX_ROWS, SUB)
    dst = edge_index[1].astype(jnp.int32).reshape(IDX_ROWS, SUB)
    zeros = jnp.zeros((ROWS_PER_TILE, HALF), jnp.float32)

    perm = jnp.asarray(_PERM)
    h, h_store = _encode(x, W_in, W_in[:, perm])
    e_embs = _edge_emb_all(edge_attr, W_edge[:, :, perm])
    for l in range(N_LAYERS):
        agg = _gin_edge(h_store, e_embs[l], src, dst, zeros)
        h, h_store = _mlp(h, agg, eps[l], W1[l], b1[l], W2[l], b2[l],
                          W2[l][:, perm], b2[l][perm],
                          relu_out=(l < N_LAYERS - 1))
    pred, h_graph = _pool_head(h, batch_vec.astype(jnp.int32), W_pred, b_pred)
    return (pred, h_graph)


# speedup vs baseline: 1.2089x; 1.2089x over previous
import functools

import numpy as np

import jax
import jax.numpy as jnp
from jax import lax
from jax.experimental import pallas as pl
from jax.experimental.pallas import tpu as pltpu
from jax.experimental.pallas import tpu_sc as plsc

N_NODES = 10000
N_EDGES = 320000
IN_DIM = 128
EMB = 128
EDGE_DIM = 16
N_LAYERS = 3
N_GRAPHS = 64
N_CLASS = 10

PACK = EMB // 2


def _make_perms():
    a = np.zeros(PACK, np.int32)
    b = np.zeros(PACK, np.int32)
    for i in range(PACK):
        g, t = divmod(i, 16)
        a[i] = 32 * g + t
        b[i] = 32 * g + 16 + t
    return a, b


_PERM_A, _PERM_B = _make_perms()


def _pack_rows(fa, fb):
    ai = jax.lax.bitcast_convert_type(fa, jnp.int32)
    bi = jax.lax.bitcast_convert_type(fb, jnp.int32)

    def rnd(v):
        return (v + 0x7FFF + ((v >> 16) & 1)) >> 16

    return ((rnd(bi) << 16) | (rnd(ai) & 0xFFFF)).astype(jnp.int32)

SC_CORES = 2
SC_SUBCORES = 16
HALF = EMB // SC_CORES
HPACK = PACK // SC_CORES
ROWS_PER_TILE = N_NODES // SC_SUBCORES
EDGES_PER_TILE = N_EDGES // SC_SUBCORES
SUB = 100
KSUB = 2
SUPER = SUB * KSUB
N_SUPER = EDGES_PER_TILE // SUPER
IDX_ROWS = N_EDGES // SUB


def _encode_body(x_ref, w_ref, wa_ref, wb_ref, o_ref, os_ref):
    x = x_ref[...]
    o_ref[...] = jnp.dot(x, w_ref[...], preferred_element_type=jnp.float32)
    os_ref[...] = _pack_rows(
        jnp.dot(x, wa_ref[...], preferred_element_type=jnp.float32),
        jnp.dot(x, wb_ref[...], preferred_element_type=jnp.float32))


def _encode(x, w_in, w_in_a, w_in_b, interpret=False):
    bm = 2000
    return pl.pallas_call(
        _encode_body,
        grid=(N_NODES // bm,),
        in_specs=[
            pl.BlockSpec((bm, IN_DIM), lambda i: (i, 0)),
            pl.BlockSpec((IN_DIM, EMB), lambda i: (0, 0)),
            pl.BlockSpec((IN_DIM, PACK), lambda i: (0, 0)),
            pl.BlockSpec((IN_DIM, PACK), lambda i: (0, 0)),
        ],
        out_specs=[
            pl.BlockSpec((bm, EMB), lambda i: (i, 0)),
            pl.BlockSpec((bm, PACK), lambda i: (i, 0)),
        ],
        out_shape=[
            jax.ShapeDtypeStruct((N_NODES, EMB), jnp.float32),
            jax.ShapeDtypeStruct((N_NODES, PACK), jnp.int32),
        ],
        interpret=interpret,
    )(x, w_in, w_in_a, w_in_b)


def _edge_emb_body(ea_ref, wa_ref, wb_ref, o0_ref, o1_ref, o2_ref):
    ea = ea_ref[...]
    for l, o in enumerate((o0_ref, o1_ref, o2_ref)):
        o[...] = _pack_rows(
            jnp.dot(ea, wa_ref[l], preferred_element_type=jnp.float32),
            jnp.dot(ea, wb_ref[l], preferred_element_type=jnp.float32))


def _edge_emb_all(edge_attr, w_edge_a, w_edge_b, interpret=False):
    bm = 4000
    out = jax.ShapeDtypeStruct((N_EDGES, PACK), jnp.int32)
    return pl.pallas_call(
        _edge_emb_body,
        grid=(N_EDGES // bm,),
        in_specs=[
            pl.BlockSpec((bm, EDGE_DIM), lambda i: (i, 0)),
            pl.BlockSpec((N_LAYERS, EDGE_DIM, PACK), lambda i: (0, 0, 0)),
            pl.BlockSpec((N_LAYERS, EDGE_DIM, PACK), lambda i: (0, 0, 0)),
        ],
        out_specs=[pl.BlockSpec((bm, PACK), lambda i: (i, 0))] * 3,
        out_shape=[out, out, out],
        interpret=interpret,
    )(edge_attr, w_edge_a, w_edge_b)


def _mlp_body(h_ref, agg_ref, eps_ref, w1_ref, b1_ref, w2_ref, b2_ref,
              wa_ref, ba_ref, wb_ref, bb_ref, o_ref, os_ref, *, relu_out):
    z = (1.0 + eps_ref[0, 0]) * h_ref[...] + agg_ref[...]
    a = jnp.maximum(jnp.dot(z, w1_ref[...],
                            preferred_element_type=jnp.float32)
                    + b1_ref[...], 0.0)
    o = jnp.dot(a, w2_ref[...], preferred_element_type=jnp.float32) + b2_ref[...]
    oa = (jnp.dot(a, wa_ref[...], preferred_element_type=jnp.float32)
          + ba_ref[...])
    ob = (jnp.dot(a, wb_ref[...], preferred_element_type=jnp.float32)
          + bb_ref[...])
    if relu_out:
        o = jnp.maximum(o, 0.0)
        oa = jnp.maximum(oa, 0.0)
        ob = jnp.maximum(ob, 0.0)
    o_ref[...] = o
    os_ref[...] = _pack_rows(oa, ob)


def _mlp(h, agg, eps_l, w1, b1, w2, b2, w2a, b2a, w2b, b2b, relu_out,
         interpret=False):
    bm = 2000
    return pl.pallas_call(
        functools.partial(_mlp_body, relu_out=relu_out),
        grid=(N_NODES // bm,),
        in_specs=[
            pl.BlockSpec((bm, EMB), lambda i: (i, 0)),
            pl.BlockSpec((bm, EMB), lambda i: (i, 0)),
            pl.BlockSpec((1, 1), lambda i: (0, 0)),
            pl.BlockSpec((EMB, 2 * EMB), lambda i: (0, 0)),
            pl.BlockSpec((1, 2 * EMB), lambda i: (0, 0)),
            pl.BlockSpec((2 * EMB, EMB), lambda i: (0, 0)),
            pl.BlockSpec((1, EMB), lambda i: (0, 0)),
            pl.BlockSpec((2 * EMB, PACK), lambda i: (0, 0)),
            pl.BlockSpec((1, PACK), lambda i: (0, 0)),
            pl.BlockSpec((2 * EMB, PACK), lambda i: (0, 0)),
            pl.BlockSpec((1, PACK), lambda i: (0, 0)),
        ],
        out_specs=[
            pl.BlockSpec((bm, EMB), lambda i: (i, 0)),
            pl.BlockSpec((bm, PACK), lambda i: (i, 0)),
        ],
        out_shape=[
            jax.ShapeDtypeStruct((N_NODES, EMB), jnp.float32),
            jax.ShapeDtypeStruct((N_NODES, PACK), jnp.int32),
        ],
        interpret=interpret,
    )(h, agg, eps_l.reshape(1, 1), w1, b1.reshape(1, -1), w2,
      b2.reshape(1, -1), w2a, b2a.reshape(1, -1), w2b, b2b.reshape(1, -1))


def _pool_head_body(h_ref, bv_ref, wp_ref, bp_ref, pred_ref, hg_ref):
    bv = bv_ref[...]
    gids = lax.broadcasted_iota(jnp.int32, (N_GRAPHS, N_NODES), 0)
    oh = (gids == bv).astype(jnp.float32)
    sums = jnp.dot(oh, h_ref[...], preferred_element_type=jnp.float32)
    counts = jnp.sum(oh, axis=1, keepdims=True)
    hg = sums / jnp.maximum(counts, 1.0)
    hg_ref[...] = hg
    pred_ref[...] = jnp.dot(hg, wp_ref[...],
                            preferred_element_type=jnp.float32) + bp_ref[...]


def _pool_head(h, batch_vec, w_pred, b_pred, interpret=False):
    return pl.pallas_call(
        _pool_head_body,
        in_specs=[
            pl.BlockSpec((N_NODES, EMB), lambda: (0, 0)),
            pl.BlockSpec((1, N_NODES), lambda: (0, 0)),
            pl.BlockSpec((EMB, N_CLASS), lambda: (0, 0)),
            pl.BlockSpec((1, N_CLASS), lambda: (0, 0)),
        ],
        out_specs=[
            pl.BlockSpec((N_GRAPHS, N_CLASS), lambda: (0, 0)),
            pl.BlockSpec((N_GRAPHS, EMB), lambda: (0, 0)),
        ],
        out_shape=[
            jax.ShapeDtypeStruct((N_GRAPHS, N_CLASS), jnp.float32),
            jax.ShapeDtypeStruct((N_GRAPHS, EMB), jnp.float32),
        ],
        interpret=interpret,
    )(h, batch_vec.reshape(1, -1), w_pred, b_pred.reshape(1, -1))


def _gin_edge_body(h_hbm, e_hbm, src_hbm, dst_hbm, zeros_hbm, out_hbm,
                   h_sh, agg_sh,
                   src0, dst0, e0, src1, dst1, e1, ga, gb, ma, mb,
                   ldsem0, ldsem1, gsa, gsb, sca, scb):
    cid = lax.axis_index("c")
    sid = lax.axis_index("s")
    col0 = cid * HPACK
    r0 = sid * ROWS_PER_TILE
    row_base = sid * (EDGES_PER_TILE // SUB)
    e_base = sid * EDGES_PER_TILE

    def loads(c, srcb, dstb, eb, sem):
        rb = row_base + c * KSUB
        eo = e_base + c * SUPER
        pltpu.async_copy(src_hbm.at[pl.ds(rb, KSUB), :], srcb, sem)
        pltpu.async_copy(dst_hbm.at[pl.ds(rb, KSUB), :], dstb, sem)
        pltpu.async_copy(e_hbm.at[pl.ds(eo, SUPER), pl.ds(col0, HPACK)], eb,
                         sem)

    def wait_loads(srcb, dstb, eb, sem):
        pltpu.make_async_copy(src_hbm.at[pl.ds(row_base, KSUB), :], srcb,
                              sem).wait()
        pltpu.make_async_copy(dst_hbm.at[pl.ds(row_base, KSUB), :], dstb,
                              sem).wait()
        pltpu.make_async_copy(
            e_hbm.at[pl.ds(e_base, SUPER), pl.ds(col0, HPACK)], eb,
            sem).wait()

    def compute(eb, k, g, m):
        @plsc.parallel_loop(0, SUB, step=1, unroll=4)
        def _(j):
            row = k * SUB + j
            for v in range(HPACK // 16):
                sl = pl.ds(v * 16, 16)
                gv = plsc.bitcast(g[j, sl], jnp.bfloat16)
                ev = plsc.bitcast(eb[row, sl], jnp.bfloat16)
                msg = jnp.maximum(gv + ev, jnp.bfloat16(0.0))
                lo, hi = plsc.unpack(msg, format=plsc.PackFormat.INTERLEAVED)
                m[j, pl.ds(v * 32, 16)] = lo
                m[j, pl.ds(v * 32 + 16, 16)] = hi

    def process(srcb, dstb, eb):
        pltpu.async_copy(h_sh.at[srcb.at[0]], ga, gsa)
        pltpu.async_copy(h_sh.at[srcb.at[1]], gb, gsb)
        pltpu.make_async_copy(h_sh.at[srcb.at[0]], ga, gsa).wait()
        compute(eb, 0, ga, ma)
        pltpu.async_copy(ma, agg_sh.at[dstb.at[0]], sca, add=True)
        pltpu.make_async_copy(h_sh.at[srcb.at[1]], gb, gsb).wait()
        compute(eb, 1, gb, mb)
        pltpu.async_copy(mb, agg_sh.at[dstb.at[1]], scb, add=True)
        pltpu.make_async_copy(ma, agg_sh.at[dstb.at[0]], sca).wait()
        pltpu.make_async_copy(mb, agg_sh.at[dstb.at[1]], scb).wait()

    pltpu.sync_copy(h_hbm.at[pl.ds(r0, ROWS_PER_TILE), pl.ds(col0, HPACK)],
                    h_sh.at[pl.ds(r0, ROWS_PER_TILE), :])
    pltpu.sync_copy(zeros_hbm, agg_sh.at[pl.ds(r0, ROWS_PER_TILE), :])

    loads(0, src0, dst0, e0, ldsem0)
    loads(1, src1, dst1, e1, ldsem1)
    plsc.subcore_barrier()

    def super_body(i, carry):
        wait_loads(src0, dst0, e0, ldsem0)
        process(src0, dst0, e0)

        @pl.when(i < N_SUPER // 2 - 1)
        def _():
            loads(2 * i + 2, src0, dst0, e0, ldsem0)

        wait_loads(src1, dst1, e1, ldsem1)
        process(src1, dst1, e1)

        @pl.when(i < N_SUPER // 2 - 1)
        def _():
            loads(2 * i + 3, src1, dst1, e1, ldsem1)

        return carry

    lax.fori_loop(0, N_SUPER // 2, super_body, 0)
    plsc.subcore_barrier()

    pltpu.sync_copy(agg_sh.at[pl.ds(r0, ROWS_PER_TILE), :],
                    out_hbm.at[pl.ds(r0, ROWS_PER_TILE),
                               pl.ds(cid * HALF, HALF)])


def _gin_edge(h, e_emb, src2d, dst2d, zeros, interpret=False):
    mesh = plsc.VectorSubcoreMesh(core_axis_name="c", subcore_axis_name="s",
                                  num_cores=SC_CORES,
                                  num_subcores=SC_SUBCORES)
    idx_t = pltpu.VMEM((KSUB, SUB), jnp.int32)
    e_t = pltpu.VMEM((SUPER, HPACK), jnp.int32)
    g_t = pltpu.VMEM((SUB, HPACK), jnp.int32)
    m_t = pltpu.VMEM((SUB, HALF), jnp.float32)
    sem = pltpu.SemaphoreType.DMA
    f = pl.kernel(
        _gin_edge_body,
        out_type=jax.ShapeDtypeStruct((N_NODES, EMB), jnp.float32),
        mesh=mesh,
        scratch_types=[
            pltpu.VMEM_SHARED((N_NODES, HPACK), jnp.int32),
            pltpu.VMEM_SHARED((N_NODES, HALF), jnp.float32),
            idx_t, idx_t, e_t,
            idx_t, idx_t, e_t,
            g_t, g_t, m_t, m_t,
            sem, sem, sem, sem, sem, sem,
        ],
        compiler_params=pltpu.CompilerParams(use_tc_tiling_on_sc=False,
                                             needs_layout_passes=False),
        interpret=interpret,
    )
    return f(h, e_emb, src2d, dst2d, zeros)


def kernel(x, edge_index, edge_attr, batch_vec, W_in, W_edge, W1, b1, W2, b2,
           eps, W_pred, b_pred):
    src = edge_index[0].astype(jnp.int32).reshape(IDX_ROWS, SUB)
    dst = edge_index[1].astype(jnp.int32).reshape(IDX_ROWS, SUB)
    zeros = jnp.zeros((ROWS_PER_TILE, HALF), jnp.float32)

    pa = jnp.asarray(_PERM_A)
    pb = jnp.asarray(_PERM_B)
    h, h_store = _encode(x, W_in, W_in[:, pa], W_in[:, pb])
    e_embs = _edge_emb_all(edge_attr, W_edge[:, :, pa], W_edge[:, :, pb])
    for l in range(N_LAYERS):
        agg = _gin_edge(h_store, e_embs[l], src, dst, zeros)
        h, h_store = _mlp(h, agg, eps[l], W1[l], b1[l], W2[l], b2[l],
                          W2[l][:, pa], b2[l][pa], W2[l][:, pb], b2[l][pb],
                          relu_out=(l < N_LAYERS - 1))
    pred, h_graph = _pool_head(h, batch_vec.astype(jnp.int32), W_pred, b_pred)
    return (pred, h_graph)


# speedup vs baseline: 1.2180x; 1.0075x over previous
import functools

import numpy as np

import jax
import jax.numpy as jnp
from jax import lax
from jax.experimental import pallas as pl
from jax.experimental.pallas import tpu as pltpu
from jax.experimental.pallas import tpu_sc as plsc

N_NODES = 10000
N_EDGES = 320000
IN_DIM = 128
EMB = 128
EDGE_DIM = 16
N_LAYERS = 3
N_GRAPHS = 64
N_CLASS = 10

PACK = EMB // 2


def _make_perms():
    a = np.zeros(PACK, np.int32)
    b = np.zeros(PACK, np.int32)
    for i in range(PACK):
        g, t = divmod(i, 16)
        a[i] = 32 * g + t
        b[i] = 32 * g + 16 + t
    return a, b


_PERM_A, _PERM_B = _make_perms()


def _pack_rows(fa, fb):
    ai = jax.lax.bitcast_convert_type(fa, jnp.int32)
    bi = jax.lax.bitcast_convert_type(fb, jnp.int32)

    def rnd(v):
        return (v + 0x7FFF + ((v >> 16) & 1)) >> 16

    return ((rnd(bi) << 16) | (rnd(ai) & 0xFFFF)).astype(jnp.int32)

SC_CORES = 2
SC_SUBCORES = 16
HALF = EMB // SC_CORES
HPACK = PACK // SC_CORES
ROWS_PER_TILE = N_NODES // SC_SUBCORES
EDGES_PER_TILE = N_EDGES // SC_SUBCORES
SUB = 100
KSUB = 2
SUPER = SUB * KSUB
N_SUPER = EDGES_PER_TILE // SUPER
IDX_ROWS = N_EDGES // SUB


def _encode_body(x_ref, w_ref, wa_ref, wb_ref, o_ref, os_ref):
    x = x_ref[...]
    o_ref[...] = jnp.dot(x, w_ref[...], preferred_element_type=jnp.float32)
    os_ref[...] = _pack_rows(
        jnp.dot(x, wa_ref[...], preferred_element_type=jnp.float32),
        jnp.dot(x, wb_ref[...], preferred_element_type=jnp.float32))


def _encode(x, w_in, w_in_a, w_in_b, interpret=False):
    bm = 2000
    return pl.pallas_call(
        _encode_body,
        grid=(N_NODES // bm,),
        in_specs=[
            pl.BlockSpec((bm, IN_DIM), lambda i: (i, 0)),
            pl.BlockSpec((IN_DIM, EMB), lambda i: (0, 0)),
            pl.BlockSpec((IN_DIM, PACK), lambda i: (0, 0)),
            pl.BlockSpec((IN_DIM, PACK), lambda i: (0, 0)),
        ],
        out_specs=[
            pl.BlockSpec((bm, EMB), lambda i: (i, 0)),
            pl.BlockSpec((bm, PACK), lambda i: (i, 0)),
        ],
        out_shape=[
            jax.ShapeDtypeStruct((N_NODES, EMB), jnp.float32),
            jax.ShapeDtypeStruct((N_NODES, PACK), jnp.int32),
        ],
        interpret=interpret,
    )(x, w_in, w_in_a, w_in_b)


def _edge_emb_body(ea_ref, wa_ref, wb_ref, o0_ref, o1_ref, o2_ref):
    ea = ea_ref[...]
    for l, o in enumerate((o0_ref, o1_ref, o2_ref)):
        o[...] = _pack_rows(
            jnp.dot(ea, wa_ref[l], preferred_element_type=jnp.float32),
            jnp.dot(ea, wb_ref[l], preferred_element_type=jnp.float32))


def _edge_emb_all(edge_attr, w_edge_a, w_edge_b, interpret=False):
    bm = 4000
    out = jax.ShapeDtypeStruct((N_EDGES, PACK), jnp.int32)
    return pl.pallas_call(
        _edge_emb_body,
        grid=(N_EDGES // bm,),
        in_specs=[
            pl.BlockSpec((bm, EDGE_DIM), lambda i: (i, 0)),
            pl.BlockSpec((N_LAYERS, EDGE_DIM, PACK), lambda i: (0, 0, 0)),
            pl.BlockSpec((N_LAYERS, EDGE_DIM, PACK), lambda i: (0, 0, 0)),
        ],
        out_specs=[pl.BlockSpec((bm, PACK), lambda i: (i, 0))] * 3,
        out_shape=[out, out, out],
        interpret=interpret,
    )(edge_attr, w_edge_a, w_edge_b)


def _mlp_body(h_ref, agg_ref, eps_ref, w1_ref, b1_ref, w2_ref, b2_ref,
              wa_ref, ba_ref, wb_ref, bb_ref, o_ref, os_ref, *, relu_out):
    z = (1.0 + eps_ref[0, 0]) * h_ref[...] + agg_ref[...]
    a = jnp.maximum(jnp.dot(z, w1_ref[...],
                            preferred_element_type=jnp.float32)
                    + b1_ref[...], 0.0)
    o = jnp.dot(a, w2_ref[...], preferred_element_type=jnp.float32) + b2_ref[...]
    oa = (jnp.dot(a, wa_ref[...], preferred_element_type=jnp.float32)
          + ba_ref[...])
    ob = (jnp.dot(a, wb_ref[...], preferred_element_type=jnp.float32)
          + bb_ref[...])
    if relu_out:
        o = jnp.maximum(o, 0.0)
        oa = jnp.maximum(oa, 0.0)
        ob = jnp.maximum(ob, 0.0)
    o_ref[...] = o
    os_ref[...] = _pack_rows(oa, ob)


def _mlp(h, agg, eps_l, w1, b1, w2, b2, w2a, b2a, w2b, b2b, relu_out,
         interpret=False):
    bm = 2000
    return pl.pallas_call(
        functools.partial(_mlp_body, relu_out=relu_out),
        grid=(N_NODES // bm,),
        in_specs=[
            pl.BlockSpec((bm, EMB), lambda i: (i, 0)),
            pl.BlockSpec((bm, EMB), lambda i: (i, 0)),
            pl.BlockSpec((1, 1), lambda i: (0, 0)),
            pl.BlockSpec((EMB, 2 * EMB), lambda i: (0, 0)),
            pl.BlockSpec((1, 2 * EMB), lambda i: (0, 0)),
            pl.BlockSpec((2 * EMB, EMB), lambda i: (0, 0)),
            pl.BlockSpec((1, EMB), lambda i: (0, 0)),
            pl.BlockSpec((2 * EMB, PACK), lambda i: (0, 0)),
            pl.BlockSpec((1, PACK), lambda i: (0, 0)),
            pl.BlockSpec((2 * EMB, PACK), lambda i: (0, 0)),
            pl.BlockSpec((1, PACK), lambda i: (0, 0)),
        ],
        out_specs=[
            pl.BlockSpec((bm, EMB), lambda i: (i, 0)),
            pl.BlockSpec((bm, PACK), lambda i: (i, 0)),
        ],
        out_shape=[
            jax.ShapeDtypeStruct((N_NODES, EMB), jnp.float32),
            jax.ShapeDtypeStruct((N_NODES, PACK), jnp.int32),
        ],
        interpret=interpret,
    )(h, agg, eps_l.reshape(1, 1), w1, b1.reshape(1, -1), w2,
      b2.reshape(1, -1), w2a, b2a.reshape(1, -1), w2b, b2b.reshape(1, -1))


def _mlp_pool_body(h_ref, agg_ref, eps_ref, w1_ref, b1_ref, w2_ref, b2_ref,
                   bv_ref, wp_ref, bp_ref, pred_ref, hg_ref):
    z = (1.0 + eps_ref[0, 0]) * h_ref[...] + agg_ref[...]
    a = jnp.maximum(jnp.dot(z, w1_ref[...],
                            preferred_element_type=jnp.float32)
                    + b1_ref[...], 0.0)
    o = jnp.dot(a, w2_ref[...], preferred_element_type=jnp.float32) + b2_ref[...]
    bv = bv_ref[...]
    gids = lax.broadcasted_iota(jnp.int32, (N_GRAPHS, N_NODES), 0)
    oh = (gids == bv).astype(jnp.float32)
    sums = jnp.dot(oh, o, preferred_element_type=jnp.float32)
    counts = jnp.sum(oh, axis=1, keepdims=True)
    hg = sums / jnp.maximum(counts, 1.0)
    hg_ref[...] = hg
    pred_ref[...] = jnp.dot(hg, wp_ref[...],
                            preferred_element_type=jnp.float32) + bp_ref[...]


def _mlp_pool(h, agg, eps_l, w1, b1, w2, b2, batch_vec, w_pred, b_pred,
              interpret=False):
    return pl.pallas_call(
        _mlp_pool_body,
        in_specs=[
            pl.BlockSpec((N_NODES, EMB), lambda: (0, 0)),
            pl.BlockSpec((N_NODES, EMB), lambda: (0, 0)),
            pl.BlockSpec((1, 1), lambda: (0, 0)),
            pl.BlockSpec((EMB, 2 * EMB), lambda: (0, 0)),
            pl.BlockSpec((1, 2 * EMB), lambda: (0, 0)),
            pl.BlockSpec((2 * EMB, EMB), lambda: (0, 0)),
            pl.BlockSpec((1, EMB), lambda: (0, 0)),
            pl.BlockSpec((1, N_NODES), lambda: (0, 0)),
            pl.BlockSpec((EMB, N_CLASS), lambda: (0, 0)),
            pl.BlockSpec((1, N_CLASS), lambda: (0, 0)),
        ],
        out_specs=[
            pl.BlockSpec((N_GRAPHS, N_CLASS), lambda: (0, 0)),
            pl.BlockSpec((N_GRAPHS, EMB), lambda: (0, 0)),
        ],
        out_shape=[
            jax.ShapeDtypeStruct((N_GRAPHS, N_CLASS), jnp.float32),
            jax.ShapeDtypeStruct((N_GRAPHS, EMB), jnp.float32),
        ],
        interpret=interpret,
    )(h, agg, eps_l.reshape(1, 1), w1, b1.reshape(1, -1), w2,
      b2.reshape(1, -1), batch_vec.reshape(1, -1), w_pred,
      b_pred.reshape(1, -1))


def _gin_edge_body(h_hbm, e_hbm, src_hbm, dst_hbm, zeros_hbm, out_hbm,
                   h_sh, agg_sh,
                   src0, dst0, e0, src1, dst1, e1, ga, gb, ma, mb,
                   ldsem0, ldsem1, gsa, gsb, sca, scb):
    cid = lax.axis_index("c")
    sid = lax.axis_index("s")
    col0 = cid * HPACK
    r0 = sid * ROWS_PER_TILE
    row_base = sid * (EDGES_PER_TILE // SUB)
    e_base = sid * EDGES_PER_TILE

    def loads(c, srcb, dstb, eb, sem):
        rb = row_base + c * KSUB
        eo = e_base + c * SUPER
        pltpu.async_copy(src_hbm.at[pl.ds(rb, KSUB), :], srcb, sem)
        pltpu.async_copy(dst_hbm.at[pl.ds(rb, KSUB), :], dstb, sem)
        pltpu.async_copy(e_hbm.at[pl.ds(eo, SUPER), pl.ds(col0, HPACK)], eb,
                         sem)

    def wait_loads(srcb, dstb, eb, sem):
        pltpu.make_async_copy(src_hbm.at[pl.ds(row_base, KSUB), :], srcb,
                              sem).wait()
        pltpu.make_async_copy(dst_hbm.at[pl.ds(row_base, KSUB), :], dstb,
                              sem).wait()
        pltpu.make_async_copy(
            e_hbm.at[pl.ds(e_base, SUPER), pl.ds(col0, HPACK)], eb,
            sem).wait()

    def compute(eb, k, g, m):
        @plsc.parallel_loop(0, SUB, step=1, unroll=4)
        def _(j):
            row = k * SUB + j
            for v in range(HPACK // 16):
                sl = pl.ds(v * 16, 16)
                gv = plsc.bitcast(g[j, sl], jnp.bfloat16)
                ev = plsc.bitcast(eb[row, sl], jnp.bfloat16)
                msg = jnp.maximum(gv + ev, jnp.bfloat16(0.0))
                lo, hi = plsc.unpack(msg, format=plsc.PackFormat.INTERLEAVED)
                m[j, pl.ds(v * 32, 16)] = lo
                m[j, pl.ds(v * 32 + 16, 16)] = hi

    def process(srcb, dstb, eb):
        pltpu.async_copy(h_sh.at[srcb.at[0]], ga, gsa)
        pltpu.async_copy(h_sh.at[srcb.at[1]], gb, gsb)
        pltpu.make_async_copy(h_sh.at[srcb.at[0]], ga, gsa).wait()
        compute(eb, 0, ga, ma)
        pltpu.async_copy(ma, agg_sh.at[dstb.at[0]], sca, add=True)
        pltpu.make_async_copy(h_sh.at[srcb.at[1]], gb, gsb).wait()
        compute(eb, 1, gb, mb)
        pltpu.async_copy(mb, agg_sh.at[dstb.at[1]], scb, add=True)
        pltpu.make_async_copy(ma, agg_sh.at[dstb.at[0]], sca).wait()
        pltpu.make_async_copy(mb, agg_sh.at[dstb.at[1]], scb).wait()

    pltpu.sync_copy(h_hbm.at[pl.ds(r0, ROWS_PER_TILE), pl.ds(col0, HPACK)],
                    h_sh.at[pl.ds(r0, ROWS_PER_TILE), :])
    pltpu.sync_copy(zeros_hbm, agg_sh.at[pl.ds(r0, ROWS_PER_TILE), :])

    loads(0, src0, dst0, e0, ldsem0)
    loads(1, src1, dst1, e1, ldsem1)
    plsc.subcore_barrier()

    def super_body(i, carry):
        wait_loads(src0, dst0, e0, ldsem0)
        process(src0, dst0, e0)

        @pl.when(i < N_SUPER // 2 - 1)
        def _():
            loads(2 * i + 2, src0, dst0, e0, ldsem0)

        wait_loads(src1, dst1, e1, ldsem1)
        process(src1, dst1, e1)

        @pl.when(i < N_SUPER // 2 - 1)
        def _():
            loads(2 * i + 3, src1, dst1, e1, ldsem1)

        return carry

    lax.fori_loop(0, N_SUPER // 2, super_body, 0)
    plsc.subcore_barrier()

    pltpu.sync_copy(agg_sh.at[pl.ds(r0, ROWS_PER_TILE), :],
                    out_hbm.at[pl.ds(r0, ROWS_PER_TILE),
                               pl.ds(cid * HALF, HALF)])


def _gin_edge(h, e_emb, src2d, dst2d, zeros, interpret=False):
    mesh = plsc.VectorSubcoreMesh(core_axis_name="c", subcore_axis_name="s",
                                  num_cores=SC_CORES,
                                  num_subcores=SC_SUBCORES)
    idx_t = pltpu.VMEM((KSUB, SUB), jnp.int32)
    e_t = pltpu.VMEM((SUPER, HPACK), jnp.int32)
    g_t = pltpu.VMEM((SUB, HPACK), jnp.int32)
    m_t = pltpu.VMEM((SUB, HALF), jnp.float32)
    sem = pltpu.SemaphoreType.DMA
    f = pl.kernel(
        _gin_edge_body,
        out_type=jax.ShapeDtypeStruct((N_NODES, EMB), jnp.float32),
        mesh=mesh,
        scratch_types=[
            pltpu.VMEM_SHARED((N_NODES, HPACK), jnp.int32),
            pltpu.VMEM_SHARED((N_NODES, HALF), jnp.float32),
            idx_t, idx_t, e_t,
            idx_t, idx_t, e_t,
            g_t, g_t, m_t, m_t,
            sem, sem, sem, sem, sem, sem,
        ],
        compiler_params=pltpu.CompilerParams(use_tc_tiling_on_sc=False,
                                             needs_layout_passes=False),
        interpret=interpret,
    )
    return f(h, e_emb, src2d, dst2d, zeros)


def kernel(x, edge_index, edge_attr, batch_vec, W_in, W_edge, W1, b1, W2, b2,
           eps, W_pred, b_pred):
    src = edge_index[0].astype(jnp.int32).reshape(IDX_ROWS, SUB)
    dst = edge_index[1].astype(jnp.int32).reshape(IDX_ROWS, SUB)
    zeros = jnp.zeros((ROWS_PER_TILE, HALF), jnp.float32)

    pa = jnp.asarray(_PERM_A)
    pb = jnp.asarray(_PERM_B)
    h, h_store = _encode(x, W_in, W_in[:, pa], W_in[:, pb])
    e_embs = _edge_emb_all(edge_attr, W_edge[:, :, pa], W_edge[:, :, pb])
    for l in range(N_LAYERS - 1):
        agg = _gin_edge(h_store, e_embs[l], src, dst, zeros)
        h, h_store = _mlp(h, agg, eps[l], W1[l], b1[l], W2[l], b2[l],
                          W2[l][:, pa], b2[l][pa], W2[l][:, pb], b2[l][pb],
                          relu_out=True)
    agg = _gin_edge(h_store, e_embs[N_LAYERS - 1], src, dst, zeros)
    pred, h_graph = _mlp_pool(h, agg, eps[N_LAYERS - 1], W1[N_LAYERS - 1],
                              b1[N_LAYERS - 1], W2[N_LAYERS - 1],
                              b2[N_LAYERS - 1], batch_vec.astype(jnp.int32),
                              W_pred, b_pred)
    return (pred, h_graph)
